# baseline (speedup 1.0000x reference)
import functools

import jax
import jax.numpy as jnp
from jax import lax
from jax.experimental import pallas as pl
from jax.experimental.pallas import tpu as pltpu
from jax.experimental.pallas import tpu_sc as plsc

_N = 10000
_E = 320000
_D_IN = 128
_D_H = 128
_D_OUT = 64
_B = 64

_NC = 2
_NS = 16
_NW = _NC * _NS
_K = 80
_EPW = _E // _NW
_NCHUNK = _EPW // _K
_NP = 10240
_RPT = _NP // _NS

_R = 2000
_NB = _N // _R

_sc_mesh = plsc.VectorSubcoreMesh(
    core_axis_name="c", subcore_axis_name="s", num_cores=_NC, num_subcores=_NS)


_NBD = 4


@functools.partial(
    pl.kernel,
    out_type=jax.ShapeDtypeStruct((_NC, _NP), jnp.float32),
    mesh=_sc_mesh,
    scratch_types=[
        pltpu.VMEM((_NBD, _K), jnp.int32),
        pltpu.VMEM((_K,), jnp.float32),
        pltpu.SemaphoreType.DMA,
        pltpu.SemaphoreType.DMA,
        pltpu.SemaphoreType.DMA,
        pltpu.SemaphoreType.DMA,
        pltpu.VMEM_SHARED((_NP,), jnp.float32),
    ],
)
def _sc_degree(dst_hbm, ones_hbm, zeros_hbm, out_hbm, idx_v, ones_v,
               sem0, sem1, sem2, sem3, acc):
    sems = [sem0, sem1, sem2, sem3]
    c = lax.axis_index("c")
    s = lax.axis_index("s")
    wid = s * _NC + c
    base = wid * _EPW
    pltpu.sync_copy(zeros_hbm, acc.at[pl.ds(s * _RPT, _RPT)])
    pltpu.sync_copy(ones_hbm, ones_v)
    plsc.subcore_barrier()

    def i_start(chunk, b):
        pltpu.async_copy(dst_hbm.at[pl.ds(base + chunk * _K, _K)],
                         idx_v.at[b], sems[b])

    def i_wait(chunk, b):
        pltpu.make_async_copy(dst_hbm.at[pl.ds(base + chunk * _K, _K)],
                              idx_v.at[b], sems[b]).wait()

    def s_ones(b):
        pltpu.sync_copy(ones_v, acc.at[idx_v.at[b]], add=True)

    for b in range(_NBD):
        i_start(b, b)

    def outer(jo, carry):
        bs = jo * _NBD
        for b in range(_NBD):
            i_wait(bs + b, b)
            s_ones(b)
            i_start(bs + _NBD + b, b)
        return carry

    ngrp = _NCHUNK // _NBD
    lax.fori_loop(0, ngrp - 1, outer, 0)
    for chunk in range((ngrp - 1) * _NBD, _NCHUNK):
        b = chunk % _NBD
        i_wait(chunk, b)
        s_ones(b)
        nxt = chunk + _NBD
        if nxt < _NCHUNK:
            i_start(nxt, b)
    plsc.subcore_barrier()
    pltpu.sync_copy(acc.at[pl.ds(s * _RPT, _RPT)],
                    out_hbm.at[c, pl.ds(s * _RPT, _RPT)])


_NBUF = 2
_NGRP = _NCHUNK // _NBUF


@functools.partial(
    pl.kernel,
    out_type=jax.ShapeDtypeStruct((_NC, _NP, _D_H), jnp.float32),
    mesh=_sc_mesh,
    scratch_types=[
        pltpu.VMEM((_EPW,), jnp.int32),
        pltpu.VMEM((_NBUF, _K), jnp.int32),
        pltpu.VMEM((_NBUF, _K, _D_H), jnp.float32),
        pltpu.SemaphoreType.DMA,
        pltpu.SemaphoreType.DMA,
        pltpu.VMEM_SHARED((_NP, _D_H), jnp.float32),
    ],
)
def _sc_scatter_rows(g_hbm, src_hbm, dst_hbm, zeros_hbm, out_hbm,
                     src_v, dst_v, rows_v, gsem0, gsem1, acc):
    gsems = [gsem0, gsem1]
    c = lax.axis_index("c")
    s = lax.axis_index("s")
    wid = s * _NC + c
    base = wid * _EPW
    pltpu.sync_copy(zeros_hbm, acc.at[pl.ds(s * _RPT, _RPT)])
    pltpu.sync_copy(src_hbm.at[pl.ds(base, _EPW)], src_v)
    plsc.subcore_barrier()

    def g_start(chunk, b):
        pltpu.async_copy(dst_hbm.at[pl.ds(base + chunk * _K, _K)],
                         dst_v.at[b], gsems[b])
        pltpu.async_copy(g_hbm.at[src_v.at[pl.ds(chunk * _K, _K)]],
                         rows_v.at[b], gsems[b])

    def g_wait(chunk, b):
        pltpu.make_async_copy(dst_hbm.at[pl.ds(base + chunk * _K, _K)],
                              dst_v.at[b], gsems[b]).wait()
        pltpu.make_async_copy(g_hbm.at[src_v.at[pl.ds(chunk * _K, _K)]],
                              rows_v.at[b], gsems[b]).wait()

    def s_sync(chunk, b):
        pltpu.sync_copy(rows_v.at[b], acc.at[dst_v.at[b]], add=True)

    for b in range(_NBUF):
        g_start(b, b)

    def outer(jo, carry):
        base = jo * _NBUF
        for b in range(_NBUF):
            g_wait(base + b, b)
            s_sync(base + b, b)
            g_start(base + _NBUF + b, b)
        return carry

    lax.fori_loop(0, _NGRP - 1, outer, 0)
    for chunk in range((_NGRP - 1) * _NBUF, _NCHUNK):
        b = chunk % _NBUF
        g_wait(chunk, b)
        s_sync(chunk, b)
        nxt = chunk + _NBUF
        if nxt < _NCHUNK:
            g_start(nxt, b)
    plsc.subcore_barrier()
    pltpu.sync_copy(acc.at[pl.ds(s * _RPT, _RPT)],
                    out_hbm.at[c, pl.ds(s * _RPT, _RPT)])


def _selu(v):
    return 1.0507009873554805 * jnp.where(
        v > 0.0, v, 1.6732632423543772 * (jnp.exp(v) - 1.0))


def _tc_h1_body(x_ref, w_ref, h_ref):
    h_ref[...] = jnp.dot(x_ref[...], w_ref[...],
                         preferred_element_type=jnp.float32)


_tc_h1 = pl.pallas_call(
    _tc_h1_body,
    grid=(_NB,),
    in_specs=[
        pl.BlockSpec((_R, _D_IN), lambda i: (i, 0)),
        pl.BlockSpec((_D_IN, _D_H), lambda i: (0, 0)),
    ],
    out_specs=pl.BlockSpec((_R, _D_H), lambda i: (i, 0)),
    out_shape=jax.ShapeDtypeStruct((_N, _D_H), jnp.float32),
)


def _tc_scale_body(deg_ref, h_ref, g_ref, dis_ref):
    dis = lax.rsqrt(deg_ref[0] + deg_ref[1] + 1.0)
    dis_ref[...] = dis
    g_ref[...] = dis * h_ref[...]


_tc_scale = pl.pallas_call(
    _tc_scale_body,
    grid=(_NB,),
    in_specs=[
        pl.BlockSpec((2, _R, 1), lambda i: (0, i, 0)),
        pl.BlockSpec((_R, _D_H), lambda i: (i, 0)),
    ],
    out_specs=[
        pl.BlockSpec((_R, _D_H), lambda i: (i, 0)),
        pl.BlockSpec((_R, 1), lambda i: (i, 0)),
    ],
    out_shape=[
        jax.ShapeDtypeStruct((_N, _D_H), jnp.float32),
        jax.ShapeDtypeStruct((_N, 1), jnp.float32),
    ],
)


def _tc_mid_body(acc_ref, g_ref, dis_ref, b_ref, w_ref, g2_ref):
    dis = dis_ref[...]
    z = _selu(dis * (acc_ref[0] + acc_ref[1] + g_ref[...]) + b_ref[...])
    g2_ref[...] = dis * jnp.dot(z, w_ref[...],
                                preferred_element_type=jnp.float32)


_tc_mid = pl.pallas_call(
    _tc_mid_body,
    grid=(_NB,),
    in_specs=[
        pl.BlockSpec((2, _R, _D_H), lambda i: (0, i, 0)),
        pl.BlockSpec((_R, _D_H), lambda i: (i, 0)),
        pl.BlockSpec((_R, 1), lambda i: (i, 0)),
        pl.BlockSpec((1, _D_H), lambda i: (0, 0)),
        pl.BlockSpec((_D_H, _D_H), lambda i: (0, 0)),
    ],
    out_specs=pl.BlockSpec((_R, _D_H), lambda i: (i, 0)),
    out_shape=jax.ShapeDtypeStruct((_N, _D_H), jnp.float32),
)


def _tc_final_body(acc_ref, g_ref, dis_ref, b_ref, batch_ref, wd_ref, bd_ref,
                   out_ref, sums_ref, cnts_ref):
    i = pl.program_id(0)

    @pl.when(i == 0)
    def _():
        sums_ref[...] = jnp.zeros_like(sums_ref)
        cnts_ref[...] = jnp.zeros_like(cnts_ref)

    dis = dis_ref[...]
    z = _selu(dis * (acc_ref[0] + acc_ref[1] + g_ref[...]) + b_ref[...])
    onehot = (batch_ref[...] ==
              lax.broadcasted_iota(jnp.int32, (1, _B), 1)).astype(jnp.float32)
    sums_ref[...] += lax.dot_general(
        onehot, z, (((0,), (0,)), ((), ())),
        preferred_element_type=jnp.float32)
    cnts_ref[...] += lax.dot_general(
        onehot, jnp.ones((_R, 1), jnp.float32), (((0,), (0,)), ((), ())),
        preferred_element_type=jnp.float32)

    @pl.when(i == _NB - 1)
    def _():
        pooled = sums_ref[...] / jnp.maximum(cnts_ref[...], 1.0)
        logits = jnp.dot(pooled, wd_ref[...],
                         preferred_element_type=jnp.float32) + bd_ref[...]
        out_ref[...] = jax.nn.sigmoid(logits)


_tc_final = pl.pallas_call(
    _tc_final_body,
    grid=(_NB,),
    in_specs=[
        pl.BlockSpec((2, _R, _D_H), lambda i: (0, i, 0)),
        pl.BlockSpec((_R, _D_H), lambda i: (i, 0)),
        pl.BlockSpec((_R, 1), lambda i: (i, 0)),
        pl.BlockSpec((1, _D_H), lambda i: (0, 0)),
        pl.BlockSpec((_R, 1), lambda i: (i, 0)),
        pl.BlockSpec((_D_H, _D_OUT), lambda i: (0, 0)),
        pl.BlockSpec((1, _D_OUT), lambda i: (0, 0)),
    ],
    out_specs=pl.BlockSpec((_B, _D_OUT), lambda i: (0, 0)),
    out_shape=jax.ShapeDtypeStruct((_B, _D_OUT), jnp.float32),
    scratch_shapes=[
        pltpu.VMEM((_B, _D_H), jnp.float32),
        pltpu.VMEM((_B, 1), jnp.float32),
    ],
)


def kernel(x, W1, b1, W2, b2, Wd, bd, edge_index, batch):
    src = edge_index[0]
    dst = edge_index[1]
    ones_k = jnp.ones((_K,), jnp.float32)
    zeros_1d = jnp.zeros((_RPT,), jnp.float32)
    zeros_2d = jnp.zeros((_RPT, _D_H), jnp.float32)
    b1r = b1.reshape(1, _D_H)
    b2r = b2.reshape(1, _D_H)
    bdr = bd.reshape(1, _D_OUT)
    batch_col = batch.reshape(_N, 1)

    deg_parts = _sc_degree(dst, ones_k, zeros_1d)
    h1 = _tc_h1(x, W1)
    deg3 = deg_parts[:, :_N].reshape(_NC, _N, 1)

    g1, dis = _tc_scale(deg3, h1)
    acc1 = _sc_scatter_rows(g1, src, dst, zeros_2d)
    g2 = _tc_mid(acc1, g1, dis, b1r, W2)
    acc2 = _sc_scatter_rows(g2, src, dst, zeros_2d)
    out = _tc_final(acc2, g2, dis, b2r, batch_col, Wd, bdr)
    return out


# speedup vs baseline: 1.0245x; 1.0245x over previous
import functools

import jax
import jax.numpy as jnp
from jax import lax
from jax.experimental import pallas as pl
from jax.experimental.pallas import tpu as pltpu
from jax.experimental.pallas import tpu_sc as plsc

_N = 10000
_E = 320000
_D_IN = 128
_D_H = 128
_D_OUT = 64
_B = 64

_NC = 2
_NS = 16
_NW = _NC * _NS
_K = 80
_EPW = _E // _NW
_NCHUNK = _EPW // _K
_NP = 10240
_RPT = _NP // _NS

_R = 2000
_NB = _N // _R

_sc_mesh = plsc.VectorSubcoreMesh(
    core_axis_name="c", subcore_axis_name="s", num_cores=_NC, num_subcores=_NS)


@functools.partial(
    pl.kernel,
    out_type=jax.ShapeDtypeStruct((_NC, _NP), jnp.float32),
    mesh=_sc_mesh,
    scratch_types=[
        pltpu.VMEM((_NCHUNK, _K), jnp.int32),
        pltpu.VMEM((_K,), jnp.float32),
        pltpu.VMEM_SHARED((_NP,), jnp.float32),
    ],
)
def _sc_degree(dst_hbm, ones_hbm, zeros_hbm, out_hbm, idx_v, ones_v, acc):
    c = lax.axis_index("c")
    s = lax.axis_index("s")
    wid = s * _NC + c
    pltpu.sync_copy(zeros_hbm, acc.at[pl.ds(s * _RPT, _RPT)])
    pltpu.sync_copy(dst_hbm.at[wid], idx_v)
    pltpu.sync_copy(ones_hbm, ones_v)
    plsc.subcore_barrier()

    def body(j, carry):
        pltpu.sync_copy(ones_v, acc.at[idx_v.at[j]], add=True)
        return carry

    lax.fori_loop(0, _NCHUNK, body, 0)
    plsc.subcore_barrier()
    pltpu.sync_copy(acc.at[pl.ds(s * _RPT, _RPT)],
                    out_hbm.at[c, pl.ds(s * _RPT, _RPT)])


_NBUF = 2
_NGRP = _NCHUNK // _NBUF


@functools.partial(
    pl.kernel,
    out_type=jax.ShapeDtypeStruct((_NC, _NP, _D_H), jnp.float32),
    mesh=_sc_mesh,
    scratch_types=[
        pltpu.VMEM((_EPW,), jnp.int32),
        pltpu.VMEM((_NBUF, _K), jnp.int32),
        pltpu.VMEM((_NBUF, _K, _D_H), jnp.float32),
        pltpu.SemaphoreType.DMA,
        pltpu.SemaphoreType.DMA,
        pltpu.VMEM_SHARED((_NP, _D_H), jnp.float32),
    ],
)
def _sc_scatter_rows(g_hbm, src_hbm, dst_hbm, zeros_hbm, out_hbm,
                     src_v, dst_v, rows_v, gsem0, gsem1, acc):
    gsems = [gsem0, gsem1]
    c = lax.axis_index("c")
    s = lax.axis_index("s")
    wid = s * _NC + c
    base = wid * _EPW
    pltpu.sync_copy(zeros_hbm, acc.at[pl.ds(s * _RPT, _RPT)])
    pltpu.sync_copy(src_hbm.at[pl.ds(base, _EPW)], src_v)
    plsc.subcore_barrier()

    def g_start(chunk, b):
        pltpu.async_copy(dst_hbm.at[pl.ds(base + chunk * _K, _K)],
                         dst_v.at[b], gsems[b])
        pltpu.async_copy(g_hbm.at[src_v.at[pl.ds(chunk * _K, _K)]],
                         rows_v.at[b], gsems[b])

    def g_wait(chunk, b):
        pltpu.make_async_copy(dst_hbm.at[pl.ds(base + chunk * _K, _K)],
                              dst_v.at[b], gsems[b]).wait()
        pltpu.make_async_copy(g_hbm.at[src_v.at[pl.ds(chunk * _K, _K)]],
                              rows_v.at[b], gsems[b]).wait()

    def s_sync(chunk, b):
        pltpu.sync_copy(rows_v.at[b], acc.at[dst_v.at[b]], add=True)

    for b in range(_NBUF):
        g_start(b, b)

    def outer(jo, carry):
        base = jo * _NBUF
        for b in range(_NBUF):
            g_wait(base + b, b)
            s_sync(base + b, b)
            g_start(base + _NBUF + b, b)
        return carry

    lax.fori_loop(0, _NGRP - 1, outer, 0)
    for chunk in range((_NGRP - 1) * _NBUF, _NCHUNK):
        b = chunk % _NBUF
        g_wait(chunk, b)
        s_sync(chunk, b)
        nxt = chunk + _NBUF
        if nxt < _NCHUNK:
            g_start(nxt, b)
    plsc.subcore_barrier()
    pltpu.sync_copy(acc.at[pl.ds(s * _RPT, _RPT)],
                    out_hbm.at[c, pl.ds(s * _RPT, _RPT)])


def _selu(v):
    return 1.0507009873554805 * jnp.where(
        v > 0.0, v, 1.6732632423543772 * (jnp.exp(v) - 1.0))


def _tc_h1_body(x_ref, w_ref, h_ref):
    h_ref[...] = jnp.dot(x_ref[...], w_ref[...],
                         preferred_element_type=jnp.float32)


_tc_h1 = pl.pallas_call(
    _tc_h1_body,
    grid=(_NB,),
    in_specs=[
        pl.BlockSpec((_R, _D_IN), lambda i: (i, 0)),
        pl.BlockSpec((_D_IN, _D_H), lambda i: (0, 0)),
    ],
    out_specs=pl.BlockSpec((_R, _D_H), lambda i: (i, 0)),
    out_shape=jax.ShapeDtypeStruct((_N, _D_H), jnp.float32),
)


def _tc_scale_body(deg_ref, h_ref, g_ref, dis_ref):
    dis = lax.rsqrt(deg_ref[...] + 1.0)
    dis_ref[...] = dis
    g_ref[...] = dis * h_ref[...]


_tc_scale = pl.pallas_call(
    _tc_scale_body,
    grid=(_NB,),
    in_specs=[
        pl.BlockSpec((_R, 1), lambda i: (i, 0)),
        pl.BlockSpec((_R, _D_H), lambda i: (i, 0)),
    ],
    out_specs=[
        pl.BlockSpec((_R, _D_H), lambda i: (i, 0)),
        pl.BlockSpec((_R, 1), lambda i: (i, 0)),
    ],
    out_shape=[
        jax.ShapeDtypeStruct((_N, _D_H), jnp.float32),
        jax.ShapeDtypeStruct((_N, 1), jnp.float32),
    ],
)


def _tc_mid_body(acc_ref, g_ref, dis_ref, b_ref, w_ref, g2_ref):
    dis = dis_ref[...]
    z = _selu(dis * (acc_ref[0] + acc_ref[1] + g_ref[...]) + b_ref[...])
    g2_ref[...] = dis * jnp.dot(z, w_ref[...],
                                preferred_element_type=jnp.float32)


_tc_mid = pl.pallas_call(
    _tc_mid_body,
    grid=(_NB,),
    in_specs=[
        pl.BlockSpec((2, _R, _D_H), lambda i: (0, i, 0)),
        pl.BlockSpec((_R, _D_H), lambda i: (i, 0)),
        pl.BlockSpec((_R, 1), lambda i: (i, 0)),
        pl.BlockSpec((1, _D_H), lambda i: (0, 0)),
        pl.BlockSpec((_D_H, _D_H), lambda i: (0, 0)),
    ],
    out_specs=pl.BlockSpec((_R, _D_H), lambda i: (i, 0)),
    out_shape=jax.ShapeDtypeStruct((_N, _D_H), jnp.float32),
)


def _tc_final_body(acc_ref, g_ref, dis_ref, b_ref, batch_ref, wd_ref, bd_ref,
                   out_ref, sums_ref, cnts_ref):
    i = pl.program_id(0)

    @pl.when(i == 0)
    def _():
        sums_ref[...] = jnp.zeros_like(sums_ref)
        cnts_ref[...] = jnp.zeros_like(cnts_ref)

    dis = dis_ref[...]
    z = _selu(dis * (acc_ref[0] + acc_ref[1] + g_ref[...]) + b_ref[...])
    onehot = (batch_ref[...] ==
              lax.broadcasted_iota(jnp.int32, (1, _B), 1)).astype(jnp.float32)
    sums_ref[...] += lax.dot_general(
        onehot, z, (((0,), (0,)), ((), ())),
        preferred_element_type=jnp.float32)
    cnts_ref[...] += lax.dot_general(
        onehot, jnp.ones((_R, 1), jnp.float32), (((0,), (0,)), ((), ())),
        preferred_element_type=jnp.float32)

    @pl.when(i == _NB - 1)
    def _():
        pooled = sums_ref[...] / jnp.maximum(cnts_ref[...], 1.0)
        logits = jnp.dot(pooled, wd_ref[...],
                         preferred_element_type=jnp.float32) + bd_ref[...]
        out_ref[...] = jax.nn.sigmoid(logits)


_tc_final = pl.pallas_call(
    _tc_final_body,
    grid=(_NB,),
    in_specs=[
        pl.BlockSpec((2, _R, _D_H), lambda i: (0, i, 0)),
        pl.BlockSpec((_R, _D_H), lambda i: (i, 0)),
        pl.BlockSpec((_R, 1), lambda i: (i, 0)),
        pl.BlockSpec((1, _D_H), lambda i: (0, 0)),
        pl.BlockSpec((_R, 1), lambda i: (i, 0)),
        pl.BlockSpec((_D_H, _D_OUT), lambda i: (0, 0)),
        pl.BlockSpec((1, _D_OUT), lambda i: (0, 0)),
    ],
    out_specs=pl.BlockSpec((_B, _D_OUT), lambda i: (0, 0)),
    out_shape=jax.ShapeDtypeStruct((_B, _D_OUT), jnp.float32),
    scratch_shapes=[
        pltpu.VMEM((_B, _D_H), jnp.float32),
        pltpu.VMEM((_B, 1), jnp.float32),
    ],
)


def kernel(x, W1, b1, W2, b2, Wd, bd, edge_index, batch):
    src = edge_index[0]
    dst = edge_index[1]
    dst3 = dst.reshape(_NW, _NCHUNK, _K)
    ones_k = jnp.ones((_K,), jnp.float32)
    zeros_1d = jnp.zeros((_RPT,), jnp.float32)
    zeros_2d = jnp.zeros((_RPT, _D_H), jnp.float32)
    b1r = b1.reshape(1, _D_H)
    b2r = b2.reshape(1, _D_H)
    bdr = bd.reshape(1, _D_OUT)
    batch_col = batch.reshape(_N, 1)

    deg_parts = _sc_degree(dst3, ones_k, zeros_1d)
    h1 = _tc_h1(x, W1)
    degsum = (deg_parts[0, :_N] + deg_parts[1, :_N]).reshape(_N, 1)

    g1, dis = _tc_scale(degsum, h1)
    acc1 = _sc_scatter_rows(g1, src, dst, zeros_2d)
    g2 = _tc_mid(acc1, g1, dis, b1r, W2)
    acc2 = _sc_scatter_rows(g2, src, dst, zeros_2d)
    out = _tc_final(acc2, g2, dis, b2r, batch_col, Wd, bdr)
    return out
